# speedup vs baseline: 1.9588x; 1.9588x over previous
import functools

import jax
import jax.numpy as jnp
from jax import lax
from jax.experimental import pallas as pl
from jax.experimental.pallas import tpu as pltpu
from jax.experimental.pallas import tpu_sc as plsc

N_NODES = 10000
N_EDGES = 320000
D_FEAT = 128
D_EDGE = 16
D_HIDDEN = 128

_NC = 2
_NS = 16
_NW = _NC * _NS
_BPW = N_EDGES // _NW
_CHUNK = 80
_NCHUNK = _BPW // _CHUNK


def _project_nodes_body(n_ref, w2_ref, h_ref):
    h_ref[...] = jnp.dot(
        n_ref[...], w2_ref[...],
        preferred_element_type=jnp.float32,
        precision=lax.Precision.HIGHEST,
    )


def _project_nodes(n_input, w2):
    return pl.pallas_call(
        _project_nodes_body,
        out_shape=jax.ShapeDtypeStruct((N_NODES, D_HIDDEN), jnp.float32),
    )(n_input, w2)


def _gather_body(h_hbm, src_hbm, g_hbm, idx_v, rows_v, sem):
    wid = lax.axis_index("s") * _NC + lax.axis_index("c")
    base = wid * _BPW
    pltpu.sync_copy(src_hbm.at[pl.ds(base, _BPW)], idx_v)

    def chunk(c, carry):
        off = c * _CHUNK
        pltpu.async_copy(
            h_hbm.at[idx_v.at[pl.ds(off, _CHUNK)]], rows_v, sem
        ).wait()
        pltpu.sync_copy(rows_v, g_hbm.at[pl.ds(base + off, _CHUNK)])
        return carry

    lax.fori_loop(0, _NCHUNK, chunk, 0)


def _gather_rows(h, src):
    mesh = plsc.VectorSubcoreMesh(
        core_axis_name="c", subcore_axis_name="s",
        num_cores=_NC, num_subcores=_NS,
    )
    return pl.kernel(
        _gather_body,
        out_type=jax.ShapeDtypeStruct((N_EDGES, D_HIDDEN), jnp.float32),
        mesh=mesh,
        scratch_types=[
            pltpu.VMEM((_BPW,), jnp.int32),
            pltpu.VMEM((_CHUNK, D_HIDDEN), jnp.float32),
            pltpu.SemaphoreType.DMA,
        ],
    )(h, src)


def _combine_body(e_ref, g_ref, w1_ref, out_ref):
    out_ref[...] = g_ref[...] + jnp.dot(
        e_ref[...], w1_ref[...], preferred_element_type=jnp.float32
    )


_ROWS = 2000


def _combine(e_input, g, w1):
    grid = (N_EDGES // _ROWS,)
    return pl.pallas_call(
        _combine_body,
        grid=grid,
        in_specs=[
            pl.BlockSpec((_ROWS, D_EDGE), lambda i: (i, 0)),
            pl.BlockSpec((_ROWS, D_HIDDEN), lambda i: (i, 0)),
            pl.BlockSpec((D_EDGE, D_HIDDEN), lambda i: (0, 0)),
        ],
        out_specs=pl.BlockSpec((_ROWS, D_HIDDEN), lambda i: (i, 0)),
        out_shape=jax.ShapeDtypeStruct((N_EDGES, D_HIDDEN), jnp.float32),
    )(e_input, g, w1)


def kernel(n_input, e_input, edge_index, W):
    src = edge_index[0].astype(jnp.int32)
    w1 = W[:D_EDGE]
    w2 = W[D_EDGE:]
    h = _project_nodes(n_input, w2)
    g = _gather_rows(h, src)
    out = _combine(e_input, g, w1)
    return out, out


# speedup vs baseline: 2.7072x; 1.3821x over previous
import jax
import jax.numpy as jnp
from jax import lax
from jax.experimental import pallas as pl
from jax.experimental.pallas import tpu as pltpu
from jax.experimental.pallas import tpu_sc as plsc

N_NODES = 10000
N_EDGES = 320000
D_FEAT = 128
D_EDGE = 16
D_HIDDEN = 128

_NC = 2
_NS = 16
_NW = _NC * _NS
_BPW = N_EDGES // _NW
_CHUNK = 80
_DEPTH = 5
_NGROUP = _BPW // (_CHUNK * _DEPTH)


def _project_nodes_body(n_ref, w_ref, h_ref):
    h_ref[...] = jnp.dot(
        n_ref[...], w_ref[D_EDGE:, :],
        preferred_element_type=jnp.float32,
        precision=lax.Precision.HIGHEST,
    )


def _project_nodes(n_input, w):
    return pl.pallas_call(
        _project_nodes_body,
        out_shape=jax.ShapeDtypeStruct((N_NODES, D_HIDDEN), jnp.float32),
    )(n_input, w)


def _gather_body(h_hbm, ei_hbm, g_hbm, idx_v, rows, gsems, ssems):
    wid = lax.axis_index("s") * _NC + lax.axis_index("c")
    base = wid * _BPW
    pltpu.sync_copy(ei_hbm.at[pl.ds(base, _BPW)], idx_v)

    def group(gi, carry):
        c0 = gi * _CHUNK * _DEPTH
        gathers = []
        for b in range(_DEPTH):
            off = c0 + b * _CHUNK
            gathers.append(pltpu.async_copy(
                h_hbm.at[idx_v.at[pl.ds(off, _CHUNK)]], rows[b], gsems[b]))
        scatters = []
        for b in range(_DEPTH):
            off = c0 + b * _CHUNK
            gathers[b].wait()
            scatters.append(pltpu.async_copy(
                rows[b], g_hbm.at[pl.ds(base + off, _CHUNK)], ssems[b]))
        for b in range(_DEPTH):
            scatters[b].wait()
        return carry

    lax.fori_loop(0, _NGROUP, group, 0)


def _gather_rows(h, edge_index):
    mesh = plsc.VectorSubcoreMesh(
        core_axis_name="c", subcore_axis_name="s",
        num_cores=_NC, num_subcores=_NS,
    )
    return pl.kernel(
        _gather_body,
        out_type=jax.ShapeDtypeStruct((N_EDGES, D_HIDDEN), jnp.float32),
        mesh=mesh,
        scratch_types=[
            pltpu.VMEM((_BPW,), jnp.int32),
            [pltpu.VMEM((_CHUNK, D_HIDDEN), jnp.float32)] * _DEPTH,
            [pltpu.SemaphoreType.DMA] * _DEPTH,
            [pltpu.SemaphoreType.DMA] * _DEPTH,
        ],
    )(h, edge_index)


def _combine_body(e_ref, g_ref, w_ref, o1_ref, o2_ref):
    s = g_ref[...] + jnp.dot(
        e_ref[...], w_ref[:D_EDGE, :], preferred_element_type=jnp.float32
    )
    o1_ref[...] = s
    o2_ref[...] = s


_ROWS = 4000


def _combine(e_input, g, w):
    grid = (N_EDGES // _ROWS,)
    out_sds = jax.ShapeDtypeStruct((N_EDGES, D_HIDDEN), jnp.float32)
    out_spec = pl.BlockSpec((_ROWS, D_HIDDEN), lambda i: (i, 0))
    return pl.pallas_call(
        _combine_body,
        grid=grid,
        in_specs=[
            pl.BlockSpec((_ROWS, D_EDGE), lambda i: (i, 0)),
            pl.BlockSpec((_ROWS, D_HIDDEN), lambda i: (i, 0)),
            pl.BlockSpec((D_EDGE + D_FEAT, D_HIDDEN), lambda i: (0, 0)),
        ],
        out_specs=(out_spec, out_spec),
        out_shape=(out_sds, out_sds),
    )(e_input, g, w)


def kernel(n_input, e_input, edge_index, W):
    ei_flat = edge_index.astype(jnp.int32).reshape(-1)
    h = _project_nodes(n_input, W)
    g = _gather_rows(h, ei_flat)
    o1, o2 = _combine(e_input, g, W)
    return o1, o2


# speedup vs baseline: 2.9240x; 1.0801x over previous
import jax
import jax.numpy as jnp
from jax import lax
from jax.experimental import pallas as pl
from jax.experimental.pallas import tpu as pltpu
from jax.experimental.pallas import tpu_sc as plsc

N_NODES = 10000
N_EDGES = 320000
D_FEAT = 128
D_EDGE = 16
D_HIDDEN = 128

_NC = 2
_NS = 16
_NW = _NC * _NS
_BPW = N_EDGES // _NW
_CHUNK = 80
_DEPTH = 5
_NGROUP = _BPW // (_CHUNK * _DEPTH)


def _project_nodes_body(n_ref, w_ref, h_ref):
    h_ref[...] = jnp.dot(
        n_ref[...], w_ref[D_EDGE:, :],
        preferred_element_type=jnp.float32,
        precision=lax.Precision.HIGHEST,
    )


def _project_nodes(n_input, w):
    return pl.pallas_call(
        _project_nodes_body,
        out_shape=jax.ShapeDtypeStruct((N_NODES, D_HIDDEN), jnp.float32),
    )(n_input, w)


def _gather_body(h_hbm, ei_hbm, g_hbm, idx_v, rows, gsems, ssems):
    wid = lax.axis_index("s") * _NC + lax.axis_index("c")
    base = wid * _BPW
    pltpu.sync_copy(ei_hbm.at[pl.ds(base, _BPW)], idx_v)

    def group(gi, carry):
        c0 = gi * _CHUNK * _DEPTH
        gathers = []
        for b in range(_DEPTH):
            off = c0 + b * _CHUNK
            gathers.append(pltpu.async_copy(
                h_hbm.at[idx_v.at[pl.ds(off, _CHUNK)]], rows[b], gsems[b]))
        scatters = []
        for b in range(_DEPTH):
            off = c0 + b * _CHUNK
            gathers[b].wait()
            scatters.append(pltpu.async_copy(
                rows[b], g_hbm.at[pl.ds(base + off, _CHUNK)], ssems[b]))
        for b in range(_DEPTH):
            scatters[b].wait()
        return carry

    lax.fori_loop(0, _NGROUP, group, 0)


def _gather_rows(h, edge_index):
    mesh = plsc.VectorSubcoreMesh(
        core_axis_name="c", subcore_axis_name="s",
        num_cores=_NC, num_subcores=_NS,
    )
    return pl.kernel(
        _gather_body,
        out_type=jax.ShapeDtypeStruct((N_EDGES, D_HIDDEN), jnp.float32),
        mesh=mesh,
        scratch_types=[
            pltpu.VMEM((_BPW,), jnp.int32),
            [pltpu.VMEM((_CHUNK, D_HIDDEN), jnp.float32)] * _DEPTH,
            [pltpu.SemaphoreType.DMA] * _DEPTH,
            [pltpu.SemaphoreType.DMA] * _DEPTH,
        ],
    )(h, edge_index)


_PACK = 8
_ROWS = 6400
_PROWS = _ROWS // _PACK


def _combine_body(ep_ref, g_ref, w_ref, o1_ref, o2_ref):
    t = jnp.dot(ep_ref[...], w_ref[...], preferred_element_type=jnp.float32)
    s = g_ref[...] + t.reshape(_ROWS, D_HIDDEN)
    o1_ref[...] = s
    o2_ref[...] = s


def _combine(e_packed, g, w1bd):
    grid = (N_EDGES // _ROWS,)
    out_sds = jax.ShapeDtypeStruct((N_EDGES, D_HIDDEN), jnp.float32)
    out_spec = pl.BlockSpec((_ROWS, D_HIDDEN), lambda i: (i, 0))
    return pl.pallas_call(
        _combine_body,
        grid=grid,
        in_specs=[
            pl.BlockSpec((_PROWS, D_FEAT), lambda i: (i, 0)),
            pl.BlockSpec((_ROWS, D_HIDDEN), lambda i: (i, 0)),
            pl.BlockSpec((D_FEAT, _PACK * D_HIDDEN), lambda i: (0, 0)),
        ],
        out_specs=(out_spec, out_spec),
        out_shape=(out_sds, out_sds),
    )(e_packed, g, w1bd)


def kernel(n_input, e_input, edge_index, W):
    ei_flat = edge_index.astype(jnp.int32).reshape(-1)
    e_packed = e_input.reshape(N_EDGES // _PACK, _PACK * D_EDGE)
    w1bd = jnp.einsum(
        "ab,kj->akbj", jnp.eye(_PACK, dtype=W.dtype), W[:D_EDGE]
    ).reshape(_PACK * D_EDGE, _PACK * D_HIDDEN)
    h = _project_nodes(n_input, W)
    g = _gather_rows(h, ei_flat)
    o1, o2 = _combine(e_packed, g, w1bd)
    return o1, o2


# speedup vs baseline: 3.5684x; 1.2204x over previous
import jax
import jax.numpy as jnp
from jax import lax
from jax.experimental import pallas as pl
from jax.experimental.pallas import tpu as pltpu
from jax.experimental.pallas import tpu_sc as plsc

N_NODES = 10000
N_EDGES = 320000
D_FEAT = 128
D_EDGE = 16
D_HIDDEN = 128

_PHASES = 5
_PE = N_EDGES // _PHASES

_NC = 2
_NS = 16
_NW = _NC * _NS
_BPW = _PE // _NW
_CHUNK = 80
_DEPTH = 5
_NGROUP = _BPW // (_CHUNK * _DEPTH)

_ROWS = 6400
_BLKS = _PE // _ROWS


def _project_nodes_body(n_ref, w_ref, h_ref):
    h_ref[...] = jnp.dot(
        n_ref[...], w_ref[D_EDGE:, :],
        preferred_element_type=jnp.float32,
        precision=lax.Precision.HIGHEST,
    )


def _project_nodes(n_input, w):
    return pl.pallas_call(
        _project_nodes_body,
        out_shape=jax.ShapeDtypeStruct((N_NODES, D_HIDDEN), jnp.float32),
    )(n_input, w)


def _make_gather_body(phase):
    ebase = phase * _PE

    def _gather_body(h_hbm, ei_hbm, g_hbm, idx_v, rows, gsems, ssems):
        wid = lax.axis_index("s") * _NC + lax.axis_index("c")
        base = wid * _BPW
        pltpu.sync_copy(ei_hbm.at[pl.ds(ebase + base, _BPW)], idx_v)

        def group(gi, carry):
            c0 = gi * _CHUNK * _DEPTH
            gathers = []
            for b in range(_DEPTH):
                off = c0 + b * _CHUNK
                gathers.append(pltpu.async_copy(
                    h_hbm.at[idx_v.at[pl.ds(off, _CHUNK)]], rows[b], gsems[b]))
            scatters = []
            for b in range(_DEPTH):
                off = c0 + b * _CHUNK
                gathers[b].wait()
                scatters.append(pltpu.async_copy(
                    rows[b], g_hbm.at[pl.ds(base + off, _CHUNK)], ssems[b]))
            for b in range(_DEPTH):
                scatters[b].wait()
            return carry

        lax.fori_loop(0, _NGROUP, group, 0)

    return _gather_body


def _gather_rows(h, ei_flat, phase):
    mesh = plsc.VectorSubcoreMesh(
        core_axis_name="c", subcore_axis_name="s",
        num_cores=_NC, num_subcores=_NS,
    )
    return pl.kernel(
        _make_gather_body(phase),
        out_type=jax.ShapeDtypeStruct((_PE, D_HIDDEN), jnp.float32),
        mesh=mesh,
        scratch_types=[
            pltpu.VMEM((_BPW,), jnp.int32),
            [pltpu.VMEM((_CHUNK, D_HIDDEN), jnp.float32)] * _DEPTH,
            [pltpu.SemaphoreType.DMA] * _DEPTH,
            [pltpu.SemaphoreType.DMA] * _DEPTH,
        ],
    )(h, ei_flat)


def _alloc_body(o1_ref, o2_ref):
    o1_ref[...] = jnp.zeros_like(o1_ref)
    o2_ref[...] = jnp.zeros_like(o2_ref)


def _alloc_outs():
    sds = jax.ShapeDtypeStruct((N_EDGES, D_HIDDEN), jnp.float32)
    spec = pl.BlockSpec((8, D_HIDDEN), lambda i: (0, 0))
    return pl.pallas_call(
        _alloc_body,
        grid=(1,),
        out_specs=(spec, spec),
        out_shape=(sds, sds),
    )()


def _combine_body(et_ref, w_ref, g_ref, o1p_ref, o2p_ref, o1_ref, o2_ref):
    del o1p_ref, o2p_ref
    t = lax.dot_general(
        et_ref[...], w_ref[:D_EDGE, :],
        (((0,), (0,)), ((), ())),
        preferred_element_type=jnp.float32,
    )
    s = g_ref[...] + t
    o1_ref[...] = s
    o2_ref[...] = s


def _combine_phase(e_t, w, g_p, o1_prev, o2_prev, phase):
    b0 = phase * _BLKS
    out_sds = jax.ShapeDtypeStruct((N_EDGES, D_HIDDEN), jnp.float32)
    out_spec = pl.BlockSpec((_ROWS, D_HIDDEN), lambda i: (b0 + i, 0))
    any_spec = pl.BlockSpec(memory_space=pl.MemorySpace.ANY)
    return pl.pallas_call(
        _combine_body,
        grid=(_BLKS,),
        in_specs=[
            pl.BlockSpec((D_EDGE, _ROWS), lambda i: (0, b0 + i)),
            pl.BlockSpec((D_EDGE + D_FEAT, D_HIDDEN), lambda i: (0, 0)),
            pl.BlockSpec((_ROWS, D_HIDDEN), lambda i: (i, 0)),
            any_spec,
            any_spec,
        ],
        out_specs=(out_spec, out_spec),
        out_shape=(out_sds, out_sds),
        input_output_aliases={3: 0, 4: 1},
    )(e_t, w, g_p, o1_prev, o2_prev)


def kernel(n_input, e_input, edge_index, W):
    ei_flat = edge_index.astype(jnp.int32).reshape(-1)
    e_t = e_input.T
    h = _project_nodes(n_input, W)
    o1, o2 = _alloc_outs()
    for p in range(_PHASES):
        g_p = _gather_rows(h, ei_flat, p)
        o1, o2 = _combine_phase(e_t, W, g_p, o1, o2, p)
    return o1, o2


# speedup vs baseline: 4.4818x; 1.2559x over previous
import jax
import jax.numpy as jnp
from jax import lax
from jax.experimental import pallas as pl
from jax.experimental.pallas import tpu as pltpu
from jax.experimental.pallas import tpu_sc as plsc

N_NODES = 10000
N_EDGES = 320000
D_FEAT = 128
D_EDGE = 16
D_HIDDEN = 128
D_HALF = D_HIDDEN // 2

_PHASES = 5
_PE = N_EDGES // _PHASES

_NC = 2
_NS = 16
_NW = _NC * _NS
_BPW = _PE // _NW
_CHUNK = 80
_DEPTH = 5
_NGROUP = _BPW // (_CHUNK * _DEPTH)

_ROWS = 6400
_HROWS = _ROWS // 2
_BLKS = _PE // _ROWS


def _project_nodes_body(n_ref, w_ref, h_ref):
    h_ref[...] = jnp.dot(
        n_ref[...], w_ref[D_EDGE:, :],
        preferred_element_type=jnp.float32,
        precision=lax.Precision.HIGHEST,
    )


def _project_nodes(n_input, w):
    return pl.pallas_call(
        _project_nodes_body,
        out_shape=jax.ShapeDtypeStruct((N_NODES, D_HIDDEN), jnp.float32),
    )(n_input, w)


def _pack_table(h):
    lo = lax.bitcast_convert_type(
        h[:, :D_HALF].astype(jnp.bfloat16), jnp.uint16).astype(jnp.uint32)
    hi = lax.bitcast_convert_type(
        h[:, D_HALF:].astype(jnp.bfloat16), jnp.uint16).astype(jnp.uint32)
    return lax.bitcast_convert_type(lo | (hi << 16), jnp.int32)


def _make_gather_body(phase):
    ebase = phase * _PE

    def _gather_body(hw_hbm, ei_hbm, g_hbm, idx_v, rows, gsems, ssems):
        wid = lax.axis_index("s") * _NC + lax.axis_index("c")
        base = wid * _BPW
        pltpu.sync_copy(ei_hbm.at[pl.ds(ebase + base, _BPW)], idx_v)

        def chunk_dst(off):
            blk = off // _ROWS
            j = off - blk * _ROWS
            half = j // _HROWS
            row0 = blk * _HROWS + (j - half * _HROWS)
            return row0, half

        def group(gi, carry):
            c0 = gi * _CHUNK * _DEPTH
            gathers = []
            for b in range(_DEPTH):
                off = c0 + b * _CHUNK
                gathers.append(pltpu.async_copy(
                    hw_hbm.at[idx_v.at[pl.ds(off, _CHUNK)]], rows[b], gsems[b]))
            scatters = []
            for b in range(_DEPTH):
                off = base + c0 + b * _CHUNK
                row0, half = chunk_dst(off)
                gathers[b].wait()
                scatters.append(pltpu.async_copy(
                    rows[b],
                    g_hbm.at[pl.ds(row0, _CHUNK), pl.ds(half * D_HALF, D_HALF)],
                    ssems[b]))
            for b in range(_DEPTH):
                scatters[b].wait()
            return carry

        lax.fori_loop(0, _NGROUP, group, 0)

    return _gather_body


def _gather_rows(hw, ei_flat, phase):
    mesh = plsc.VectorSubcoreMesh(
        core_axis_name="c", subcore_axis_name="s",
        num_cores=_NC, num_subcores=_NS,
    )
    return pl.kernel(
        _make_gather_body(phase),
        out_type=jax.ShapeDtypeStruct((_PE // 2, D_HIDDEN), jnp.int32),
        mesh=mesh,
        compiler_params=pltpu.CompilerParams(use_tc_tiling_on_sc=False),
        scratch_types=[
            pltpu.VMEM((_BPW,), jnp.int32),
            [pltpu.VMEM((_CHUNK, D_HALF), jnp.int32)] * _DEPTH,
            [pltpu.SemaphoreType.DMA] * _DEPTH,
            [pltpu.SemaphoreType.DMA] * _DEPTH,
        ],
    )(hw, ei_flat)


def _alloc_body(o1_ref, o2_ref):
    o1_ref[...] = jnp.zeros_like(o1_ref)
    o2_ref[...] = jnp.zeros_like(o2_ref)


def _alloc_outs():
    sds = jax.ShapeDtypeStruct((N_EDGES, D_HIDDEN), jnp.float32)
    spec = pl.BlockSpec((8, D_HIDDEN), lambda i: (0, 0))
    return pl.pallas_call(
        _alloc_body,
        grid=(1,),
        out_specs=(spec, spec),
        out_shape=(sds, sds),
    )()


def _combine_body(et_ref, w_ref, gw_ref, o1p_ref, o2p_ref, o1_ref, o2_ref):
    del o1p_ref, o2p_ref
    t = lax.dot_general(
        et_ref[...], w_ref[:D_EDGE, :],
        (((0,), (0,)), ((), ())),
        preferred_element_type=jnp.float32,
    )
    w_words = gw_ref[...]
    a = lax.bitcast_convert_type(w_words << 16, jnp.float32)
    b = lax.bitcast_convert_type(
        w_words & jnp.int32(-65536), jnp.float32)
    first = jnp.concatenate([a[:, :D_HALF], b[:, :D_HALF]], axis=1)
    second = jnp.concatenate([a[:, D_HALF:], b[:, D_HALF:]], axis=1)
    g = jnp.concatenate([first, second], axis=0)
    s = g + t
    o1_ref[...] = s
    o2_ref[...] = s


def _combine_phase(e_t, w, g_p, o1_prev, o2_prev, phase):
    b0 = phase * _BLKS
    out_sds = jax.ShapeDtypeStruct((N_EDGES, D_HIDDEN), jnp.float32)
    out_spec = pl.BlockSpec((_ROWS, D_HIDDEN), lambda i: (b0 + i, 0))
    any_spec = pl.BlockSpec(memory_space=pl.MemorySpace.ANY)
    return pl.pallas_call(
        _combine_body,
        grid=(_BLKS,),
        in_specs=[
            pl.BlockSpec((D_EDGE, _ROWS), lambda i: (0, b0 + i)),
            pl.BlockSpec((D_EDGE + D_FEAT, D_HIDDEN), lambda i: (0, 0)),
            pl.BlockSpec((_HROWS, D_HIDDEN), lambda i: (i, 0)),
            any_spec,
            any_spec,
        ],
        out_specs=(out_spec, out_spec),
        out_shape=(out_sds, out_sds),
        input_output_aliases={3: 0, 4: 1},
    )(e_t, w, g_p, o1_prev, o2_prev)


def kernel(n_input, e_input, edge_index, W):
    ei_flat = edge_index.astype(jnp.int32).reshape(-1)
    e_t = e_input.T
    h = _project_nodes(n_input, W)
    hw = _pack_table(h)
    o1, o2 = _alloc_outs()
    for p in range(_PHASES):
        g_p = _gather_rows(hw, ei_flat, p)
        o1, o2 = _combine_phase(e_t, W, g_p, o1, o2, p)
    return o1, o2


# speedup vs baseline: 4.5653x; 1.0186x over previous
import jax
import jax.numpy as jnp
from jax import lax
from jax.experimental import pallas as pl
from jax.experimental.pallas import tpu as pltpu
from jax.experimental.pallas import tpu_sc as plsc

N_NODES = 10000
N_EDGES = 320000
D_FEAT = 128
D_EDGE = 16
D_HIDDEN = 128
D_HALF = D_HIDDEN // 2

_PHASES = 5
_PE = N_EDGES // _PHASES

_NC = 2
_NS = 16
_NW = _NC * _NS
_BPW = _PE // _NW
_CHUNK = 80
_DEPTH = 5
_NGROUP = _BPW // (_CHUNK * _DEPTH)

_ROWS = 6400
_HROWS = _ROWS // 2
_BLKS = _PE // _ROWS


def _project_nodes_body(n_ref, w_ref, hw_ref):
    h = jnp.dot(n_ref[...], w_ref[D_EDGE:, :],
                preferred_element_type=jnp.float32)
    lo = lax.bitcast_convert_type(
        h[:, :D_HALF].astype(jnp.bfloat16), jnp.uint16).astype(jnp.uint32)
    hi = lax.bitcast_convert_type(
        h[:, D_HALF:].astype(jnp.bfloat16), jnp.uint16).astype(jnp.uint32)
    hw_ref[...] = lax.bitcast_convert_type(lo | (hi << 16), jnp.int32)


def _project_nodes(n_input, w):
    return pl.pallas_call(
        _project_nodes_body,
        out_shape=jax.ShapeDtypeStruct((N_NODES, D_HALF), jnp.int32),
    )(n_input, w)


def _make_gather_body(phase):
    ebase = phase * _PE

    def _gather_body(hw_hbm, ei_hbm, g_hbm, idx_v, rows, gsems, ssems):
        wid = lax.axis_index("s") * _NC + lax.axis_index("c")
        base = wid * _BPW
        pltpu.sync_copy(ei_hbm.at[0, pl.ds(ebase + base, _BPW)], idx_v)

        def chunk_dst(off):
            blk = off // _ROWS
            j = off - blk * _ROWS
            half = j // _HROWS
            row0 = blk * _HROWS + (j - half * _HROWS)
            return row0, half

        def group(gi, carry):
            c0 = gi * _CHUNK * _DEPTH
            gathers = []
            for b in range(_DEPTH):
                off = c0 + b * _CHUNK
                gathers.append(pltpu.async_copy(
                    hw_hbm.at[idx_v.at[pl.ds(off, _CHUNK)]], rows[b], gsems[b]))
            scatters = []
            for b in range(_DEPTH):
                off = base + c0 + b * _CHUNK
                row0, half = chunk_dst(off)
                gathers[b].wait()
                scatters.append(pltpu.async_copy(
                    rows[b],
                    g_hbm.at[pl.ds(row0, _CHUNK), pl.ds(half * D_HALF, D_HALF)],
                    ssems[b]))
            for b in range(_DEPTH):
                scatters[b].wait()
            return carry

        lax.fori_loop(0, _NGROUP, group, 0)

    return _gather_body


def _gather_rows(hw, ei_flat, phase):
    mesh = plsc.VectorSubcoreMesh(
        core_axis_name="c", subcore_axis_name="s",
        num_cores=_NC, num_subcores=_NS,
    )
    return pl.kernel(
        _make_gather_body(phase),
        out_type=jax.ShapeDtypeStruct((_PE // 2, D_HIDDEN), jnp.int32),
        mesh=mesh,
        compiler_params=pltpu.CompilerParams(use_tc_tiling_on_sc=False),
        scratch_types=[
            pltpu.VMEM((_BPW,), jnp.int32),
            [pltpu.VMEM((_CHUNK, D_HALF), jnp.int32)] * _DEPTH,
            [pltpu.SemaphoreType.DMA] * _DEPTH,
            [pltpu.SemaphoreType.DMA] * _DEPTH,
        ],
    )(hw, ei_flat)


def _alloc_body(o1_ref, o2_ref):
    o1_ref[...] = jnp.zeros_like(o1_ref)
    o2_ref[...] = jnp.zeros_like(o2_ref)


def _alloc_outs():
    sds = jax.ShapeDtypeStruct((N_EDGES, D_HIDDEN), jnp.float32)
    spec = pl.BlockSpec((8, D_HIDDEN), lambda i: (0, 0))
    return pl.pallas_call(
        _alloc_body,
        grid=(1,),
        out_specs=(spec, spec),
        out_shape=(sds, sds),
    )()


def _combine_body(et_ref, w_ref, gw_ref, o1p_ref, o2p_ref, o1_ref, o2_ref):
    del o1p_ref, o2p_ref
    t = lax.dot_general(
        et_ref[...], w_ref[:D_EDGE, :],
        (((0,), (0,)), ((), ())),
        preferred_element_type=jnp.float32,
    )
    w_words = gw_ref[...]
    a = lax.bitcast_convert_type(w_words << 16, jnp.float32)
    b = lax.bitcast_convert_type(
        w_words & jnp.int32(-65536), jnp.float32)
    first = jnp.concatenate([a[:, :D_HALF], b[:, :D_HALF]], axis=1)
    second = jnp.concatenate([a[:, D_HALF:], b[:, D_HALF:]], axis=1)
    g = jnp.concatenate([first, second], axis=0)
    s = g + t
    o1_ref[...] = s
    o2_ref[...] = s


def _combine_phase(e_t, w, g_p, o1_prev, o2_prev, phase):
    b0 = phase * _BLKS
    out_sds = jax.ShapeDtypeStruct((N_EDGES, D_HIDDEN), jnp.float32)
    out_spec = pl.BlockSpec((_ROWS, D_HIDDEN), lambda i: (b0 + i, 0))
    any_spec = pl.BlockSpec(memory_space=pl.MemorySpace.ANY)
    return pl.pallas_call(
        _combine_body,
        grid=(_BLKS,),
        in_specs=[
            pl.BlockSpec((D_EDGE, _ROWS), lambda i: (0, b0 + i)),
            pl.BlockSpec((D_EDGE + D_FEAT, D_HIDDEN), lambda i: (0, 0)),
            pl.BlockSpec((_HROWS, D_HIDDEN), lambda i: (i, 0)),
            any_spec,
            any_spec,
        ],
        out_specs=(out_spec, out_spec),
        out_shape=(out_sds, out_sds),
        input_output_aliases={3: 0, 4: 1},
    )(e_t, w, g_p, o1_prev, o2_prev)


def kernel(n_input, e_input, edge_index, W):
    ei = edge_index.astype(jnp.int32)
    e_t = e_input.T
    hw = _project_nodes(n_input, W)
    o1, o2 = _alloc_outs()
    for p in range(_PHASES):
        g_p = _gather_rows(hw, ei, p)
        o1, o2 = _combine_phase(e_t, W, g_p, o1, o2, p)
    return o1, o2


# speedup vs baseline: 5.1107x; 1.1195x over previous
import jax
import jax.numpy as jnp
from jax import lax
from jax.experimental import pallas as pl
from jax.experimental.pallas import tpu as pltpu
from jax.experimental.pallas import tpu_sc as plsc

N_NODES = 10000
N_EDGES = 320000
D_FEAT = 128
D_EDGE = 16
D_HIDDEN = 128
D_HALF = D_HIDDEN // 2

_PHASES = 5
_PE = N_EDGES // _PHASES

_NC = 2
_NS = 16
_NW = _NC * _NS
_BPW = _PE // _NW
_CHUNK = 80
_DEPTH = 5
_NGROUP = _BPW // (_CHUNK * _DEPTH)

_ROWS = 6400
_HROWS = _ROWS // 2
_BLKS = _PE // _ROWS


def _project_nodes_body(n_ref, w_ref, hw_ref):
    h = jnp.dot(n_ref[...], w_ref[D_EDGE:, :],
                preferred_element_type=jnp.float32)
    lo = lax.bitcast_convert_type(
        h[:, :D_HALF].astype(jnp.bfloat16), jnp.uint16).astype(jnp.uint32)
    hi = lax.bitcast_convert_type(
        h[:, D_HALF:].astype(jnp.bfloat16), jnp.uint16).astype(jnp.uint32)
    hw_ref[...] = lax.bitcast_convert_type(lo | (hi << 16), jnp.int32)


def _project_nodes(n_input, w):
    return pl.pallas_call(
        _project_nodes_body,
        out_shape=jax.ShapeDtypeStruct((N_NODES, D_HALF), jnp.int32),
    )(n_input, w)


def _make_gather_body(phase):
    ebase = phase * _PE

    def _gather_body(hw_hbm, ei_hbm, g_hbm, table_s, idx_v, rows, gsems, ssems):
        sid = lax.axis_index("s")
        wid = sid * _NC + lax.axis_index("c")
        base = wid * _BPW

        @pl.when(sid == 0)
        def _load_table():
            pltpu.sync_copy(hw_hbm, table_s)

        pltpu.sync_copy(ei_hbm.at[0, pl.ds(ebase + base, _BPW)], idx_v)
        plsc.subcore_barrier()

        def chunk_dst(off):
            blk = off // _ROWS
            j = off - blk * _ROWS
            half = j // _HROWS
            row0 = blk * _HROWS + (j - half * _HROWS)
            return row0, half

        def group(gi, carry):
            c0 = gi * _CHUNK * _DEPTH
            gathers = []
            for b in range(_DEPTH):
                off = c0 + b * _CHUNK
                gathers.append(pltpu.async_copy(
                    table_s.at[idx_v.at[pl.ds(off, _CHUNK)]], rows[b], gsems[b]))
            scatters = []
            for b in range(_DEPTH):
                off = base + c0 + b * _CHUNK
                row0, half = chunk_dst(off)
                gathers[b].wait()
                scatters.append(pltpu.async_copy(
                    rows[b],
                    g_hbm.at[pl.ds(row0, _CHUNK), pl.ds(half * D_HALF, D_HALF)],
                    ssems[b]))
            for b in range(_DEPTH):
                scatters[b].wait()
            return carry

        lax.fori_loop(0, _NGROUP, group, 0)

    return _gather_body


def _gather_rows(hw, ei_flat, phase):
    mesh = plsc.VectorSubcoreMesh(
        core_axis_name="c", subcore_axis_name="s",
        num_cores=_NC, num_subcores=_NS,
    )
    return pl.kernel(
        _make_gather_body(phase),
        out_type=jax.ShapeDtypeStruct((_PE // 2, D_HIDDEN), jnp.int32),
        mesh=mesh,
        compiler_params=pltpu.CompilerParams(use_tc_tiling_on_sc=False),
        scratch_types=[
            pltpu.VMEM_SHARED((N_NODES, D_HALF), jnp.int32),
            pltpu.VMEM((_BPW,), jnp.int32),
            [pltpu.VMEM((_CHUNK, D_HALF), jnp.int32)] * _DEPTH,
            [pltpu.SemaphoreType.DMA] * _DEPTH,
            [pltpu.SemaphoreType.DMA] * _DEPTH,
        ],
    )(hw, ei_flat)


def _alloc_body(o1_ref, o2_ref):
    o1_ref[...] = jnp.zeros_like(o1_ref)
    o2_ref[...] = jnp.zeros_like(o2_ref)


def _alloc_outs():
    sds = jax.ShapeDtypeStruct((N_EDGES, D_HIDDEN), jnp.float32)
    spec = pl.BlockSpec((8, D_HIDDEN), lambda i: (0, 0))
    return pl.pallas_call(
        _alloc_body,
        grid=(1,),
        out_specs=(spec, spec),
        out_shape=(sds, sds),
    )()


def _combine_body(et_ref, w_ref, gw_ref, o1p_ref, o2p_ref, o1_ref, o2_ref):
    del o1p_ref, o2p_ref
    t = lax.dot_general(
        et_ref[...], w_ref[:D_EDGE, :],
        (((0,), (0,)), ((), ())),
        preferred_element_type=jnp.float32,
    )
    w_words = gw_ref[...]
    a = lax.bitcast_convert_type(w_words << 16, jnp.float32)
    b = lax.bitcast_convert_type(
        w_words & jnp.int32(-65536), jnp.float32)
    first = jnp.concatenate([a[:, :D_HALF], b[:, :D_HALF]], axis=1)
    second = jnp.concatenate([a[:, D_HALF:], b[:, D_HALF:]], axis=1)
    g = jnp.concatenate([first, second], axis=0)
    s = g + t
    o1_ref[...] = s
    o2_ref[...] = s


def _combine_phase(e_t, w, g_p, o1_prev, o2_prev, phase):
    b0 = phase * _BLKS
    out_sds = jax.ShapeDtypeStruct((N_EDGES, D_HIDDEN), jnp.float32)
    out_spec = pl.BlockSpec((_ROWS, D_HIDDEN), lambda i: (b0 + i, 0))
    any_spec = pl.BlockSpec(memory_space=pl.MemorySpace.ANY)
    return pl.pallas_call(
        _combine_body,
        grid=(_BLKS,),
        in_specs=[
            pl.BlockSpec((D_EDGE, _ROWS), lambda i: (0, b0 + i)),
            pl.BlockSpec((D_EDGE + D_FEAT, D_HIDDEN), lambda i: (0, 0)),
            pl.BlockSpec((_HROWS, D_HIDDEN), lambda i: (i, 0)),
            any_spec,
            any_spec,
        ],
        out_specs=(out_spec, out_spec),
        out_shape=(out_sds, out_sds),
        input_output_aliases={3: 0, 4: 1},
    )(e_t, w, g_p, o1_prev, o2_prev)


def kernel(n_input, e_input, edge_index, W):
    ei = edge_index.astype(jnp.int32)
    e_t = e_input.T
    hw = _project_nodes(n_input, W)
    o1, o2 = _alloc_outs()
    for p in range(_PHASES):
        g_p = _gather_rows(hw, ei, p)
        o1, o2 = _combine_phase(e_t, W, g_p, o1, o2, p)
    return o1, o2


# speedup vs baseline: 5.2142x; 1.0203x over previous
import jax
import jax.numpy as jnp
from jax import lax
from jax.experimental import pallas as pl
from jax.experimental.pallas import tpu as pltpu
from jax.experimental.pallas import tpu_sc as plsc

N_NODES = 10000
N_EDGES = 320000
D_FEAT = 128
D_EDGE = 16
D_HIDDEN = 128
D_HALF = D_HIDDEN // 2

_PHASES = 5
_PE = N_EDGES // _PHASES

_NC = 2
_NS = 16
_NW = _NC * _NS
_BPW = _PE // _NW
_CHUNK = 80
_DEPTH = 5
_NGROUP = _BPW // (_CHUNK * _DEPTH)

_ROWS = 12800
_HROWS = _ROWS // 2
_BLKS = _PE // _ROWS


def _project_nodes_body(n_ref, w_ref, hw_ref):
    h = jnp.dot(n_ref[...], w_ref[D_EDGE:, :],
                preferred_element_type=jnp.float32)
    lo = lax.bitcast_convert_type(
        h[:, :D_HALF].astype(jnp.bfloat16), jnp.uint16).astype(jnp.uint32)
    hi = lax.bitcast_convert_type(
        h[:, D_HALF:].astype(jnp.bfloat16), jnp.uint16).astype(jnp.uint32)
    hw_ref[...] = lax.bitcast_convert_type(lo | (hi << 16), jnp.int32)


def _project_nodes(n_input, w):
    return pl.pallas_call(
        _project_nodes_body,
        out_shape=jax.ShapeDtypeStruct((N_NODES, D_HALF), jnp.int32),
    )(n_input, w)


def _make_gather_body(phase):
    ebase = phase * _PE

    def _gather_body(hw_hbm, ei_hbm, g_hbm, table_s, idx_v, rows, gsems, ssems):
        sid = lax.axis_index("s")
        wid = sid * _NC + lax.axis_index("c")
        base = wid * _BPW

        @pl.when(sid == 0)
        def _load_table():
            pltpu.sync_copy(hw_hbm, table_s)

        pltpu.sync_copy(ei_hbm.at[0, pl.ds(ebase + base, _BPW)], idx_v)
        plsc.subcore_barrier()

        def chunk_dst(off):
            blk = off // _ROWS
            j = off - blk * _ROWS
            half = j // _HROWS
            row0 = blk * _HROWS + (j - half * _HROWS)
            return row0, half

        def group(gi, carry):
            c0 = gi * _CHUNK * _DEPTH
            gathers = []
            for b in range(_DEPTH):
                off = c0 + b * _CHUNK
                gathers.append(pltpu.async_copy(
                    table_s.at[idx_v.at[pl.ds(off, _CHUNK)]], rows[b], gsems[b]))
            scatters = []
            for b in range(_DEPTH):
                off = base + c0 + b * _CHUNK
                row0, half = chunk_dst(off)
                gathers[b].wait()
                scatters.append(pltpu.async_copy(
                    rows[b],
                    g_hbm.at[pl.ds(row0, _CHUNK), pl.ds(half * D_HALF, D_HALF)],
                    ssems[b]))
            for b in range(_DEPTH):
                scatters[b].wait()
            return carry

        lax.fori_loop(0, _NGROUP, group, 0)

    return _gather_body


def _gather_rows(hw, ei_flat, phase):
    mesh = plsc.VectorSubcoreMesh(
        core_axis_name="c", subcore_axis_name="s",
        num_cores=_NC, num_subcores=_NS,
    )
    return pl.kernel(
        _make_gather_body(phase),
        out_type=jax.ShapeDtypeStruct((_PE // 2, D_HIDDEN), jnp.int32),
        mesh=mesh,
        compiler_params=pltpu.CompilerParams(use_tc_tiling_on_sc=False),
        scratch_types=[
            pltpu.VMEM_SHARED((N_NODES, D_HALF), jnp.int32),
            pltpu.VMEM((_BPW,), jnp.int32),
            [pltpu.VMEM((_CHUNK, D_HALF), jnp.int32)] * _DEPTH,
            [pltpu.SemaphoreType.DMA] * _DEPTH,
            [pltpu.SemaphoreType.DMA] * _DEPTH,
        ],
    )(hw, ei_flat)


def _alloc_body(o1_ref, o2_ref):
    o1_ref[...] = jnp.zeros_like(o1_ref)
    o2_ref[...] = jnp.zeros_like(o2_ref)


def _alloc_outs():
    sds = jax.ShapeDtypeStruct((N_EDGES, D_HIDDEN), jnp.float32)
    spec = pl.BlockSpec((8, D_HIDDEN), lambda i: (0, 0))
    return pl.pallas_call(
        _alloc_body,
        grid=(1,),
        out_specs=(spec, spec),
        out_shape=(sds, sds),
    )()


def _combine_body(et_ref, w_ref, gw_ref, o1p_ref, o2p_ref, o1_ref, o2_ref):
    del o1p_ref, o2p_ref
    t = lax.dot_general(
        et_ref[...], w_ref[:D_EDGE, :],
        (((0,), (0,)), ((), ())),
        preferred_element_type=jnp.float32,
    )
    w_words = gw_ref[...]
    a = lax.bitcast_convert_type(w_words << 16, jnp.float32)
    b = lax.bitcast_convert_type(
        w_words & jnp.int32(-65536), jnp.float32)
    first = jnp.concatenate([a[:, :D_HALF], b[:, :D_HALF]], axis=1)
    second = jnp.concatenate([a[:, D_HALF:], b[:, D_HALF:]], axis=1)
    g = jnp.concatenate([first, second], axis=0)
    s = g + t
    o1_ref[...] = s
    o2_ref[...] = s


def _combine_phase(e_t, w, g_p, o1_prev, o2_prev, phase):
    b0 = phase * _BLKS
    out_sds = jax.ShapeDtypeStruct((N_EDGES, D_HIDDEN), jnp.float32)
    out_spec = pl.BlockSpec((_ROWS, D_HIDDEN), lambda i: (b0 + i, 0))
    any_spec = pl.BlockSpec(memory_space=pl.MemorySpace.ANY)
    return pl.pallas_call(
        _combine_body,
        grid=(_BLKS,),
        in_specs=[
            pl.BlockSpec((D_EDGE, _ROWS), lambda i: (0, b0 + i)),
            pl.BlockSpec((D_EDGE + D_FEAT, D_HIDDEN), lambda i: (0, 0)),
            pl.BlockSpec((_HROWS, D_HIDDEN), lambda i: (i, 0)),
            any_spec,
            any_spec,
        ],
        out_specs=(out_spec, out_spec),
        out_shape=(out_sds, out_sds),
        input_output_aliases={3: 0, 4: 1},
    )(e_t, w, g_p, o1_prev, o2_prev)


def kernel(n_input, e_input, edge_index, W):
    ei = edge_index.astype(jnp.int32)
    e_t = e_input.T
    hw = _project_nodes(n_input, W)
    o1, o2 = _alloc_outs()
    for p in range(_PHASES):
        g_p = _gather_rows(hw, ei, p)
        o1, o2 = _combine_phase(e_t, W, g_p, o1, o2, p)
    return o1, o2
